# baseline (speedup 1.0000x reference)
import functools

import jax
import jax.numpy as jnp
from jax.experimental import pallas as pl
from jax.experimental.pallas import tpu as pltpu

_B, _C, _H, _W = 16, 64, 32, 32
_P = _H * _W
_K = 1024
_NB = 2
_PT = _NB * _P
_G = _B // _NB


def _vq_body(x_ref, et_ref, e2_ref, q_ref, idx_ref, acc_ref,
             e3_ref, et2_ref):
    g = pl.program_id(0)
    x_cf = x_ref[0]
    e_t = et_ref[...]
    e2 = e2_ref[...]

    @pl.when(g == 0)
    def _build_e3():
        msk = jnp.uint32(0xFFFF0000)
        u = jax.lax.bitcast_convert_type(e_t, jnp.uint32)
        ea_f = jax.lax.bitcast_convert_type(u & msk, jnp.float32)
        r1 = e_t - ea_f
        v = jax.lax.bitcast_convert_type(r1, jnp.uint32)
        eb_f = jax.lax.bitcast_convert_type(v & msk, jnp.float32)
        r2 = r1 - eb_f
        kio = jax.lax.broadcasted_iota(jnp.int32, (1, _K), 1)
        aux = jnp.concatenate(
            [(kio // 256).astype(jnp.float32),
             (kio % 256).astype(jnp.float32),
             jnp.ones((1, _K), jnp.float32)], axis=0)
        e3_ref[...] = jnp.concatenate(
            [ea_f, eb_f, r2, aux], axis=0).astype(jnp.bfloat16)
        et2_ref[...] = e_t + e_t

    e3 = e3_ref[...]

    x_pm = jnp.transpose(x_cf, (0, 2, 1)).reshape(_PT, _C)
    x2 = jnp.sum(x_pm * x_pm, axis=1, keepdims=True)
    xe2 = jax.lax.dot_general(
        x_pm, et2_ref[...], (((1,), (0,)), ((), ())),
        preferred_element_type=jnp.float32)
    distances = (x2 + e2) - xe2

    def _emit(onehot_cp):
        out = jax.lax.dot_general(
            e3, onehot_cp, (((1,), (0,)), ((), ())),
            preferred_element_type=jnp.float32)
        qsum = (out[0:_C] + out[_C:2 * _C]) + out[2 * _C:3 * _C]
        idx_f = out[192:193] * 256.0 + out[193:194]
        idx_i = idx_f[0].astype(jnp.int32)
        for j in range(_NB):
            q_ref[0, j] = qsum[:, j * _P:(j + 1) * _P]
            idx_ref[0, j, :] = idx_i[j * _P:(j + 1) * _P]
        return out[194:195]

    dmin = jnp.min(distances, axis=1, keepdims=True)
    onehot = jnp.where(distances == dmin,
                       1.0, 0.0).astype(jnp.bfloat16)
    nhot = _emit(jnp.transpose(onehot, (1, 0)))

    @pl.when(jnp.max(nhot) > 1.5)
    def _tie_fixup():
        idx = jnp.argmax(-distances, axis=1).astype(jnp.int32)
        iota2 = jax.lax.broadcasted_iota(jnp.int32, (_K, _PT), 0)
        onehot1 = jnp.where(iota2 == idx[None, :],
                            1.0, 0.0).astype(jnp.bfloat16)
        _emit(onehot1)

    @pl.when(g == 0)
    def _init():
        acc_ref[...] = jnp.zeros((1, 1), jnp.float32)

    acc_ref[...] += jnp.sum(dmin).reshape(1, 1)


@functools.partial(jax.jit, static_argnames=())
def kernel(inputs, embedding_weight):
    inputs = inputs.astype(jnp.float32)
    x = inputs.reshape(_G, _NB, _C, _P)
    e_t = jnp.transpose(embedding_weight, (1, 0))
    e2 = jnp.sum(embedding_weight.T ** 2, axis=0, keepdims=True)

    q, idx3, acc = pl.pallas_call(
        _vq_body,
        grid=(_G,),
        in_specs=[
            pl.BlockSpec((1, _NB, _C, _P), lambda g: (g, 0, 0, 0)),
            pl.BlockSpec((_C, _K), lambda g: (0, 0)),
            pl.BlockSpec((1, _K), lambda g: (0, 0)),
        ],
        scratch_shapes=[pltpu.VMEM((3 * _C + 3, _K), jnp.bfloat16),
                        pltpu.VMEM((_C, _K), jnp.float32)],
        out_specs=[
            pl.BlockSpec((1, _NB, _C, _P), lambda g: (g, 0, 0, 0)),
            pl.BlockSpec((1, _NB, _P), lambda g: (g, 0, 0)),
            pl.BlockSpec((1, 1), lambda g: (0, 0)),
        ],
        out_shape=[
            jax.ShapeDtypeStruct((_G, _NB, _C, _P), jnp.float32),
            jax.ShapeDtypeStruct((_G, _NB, _P), jnp.int32),
            jax.ShapeDtypeStruct((1, 1), jnp.float32),
        ],
        compiler_params=pltpu.CompilerParams(
            dimension_semantics=("arbitrary",)),
    )(x, e_t, e2)

    quantized_st = q.reshape(_B, _C, _H, _W)
    encoding_indices_v = idx3.reshape(_B, _H, _W)
    loss = acc[0, 0] * (0.25 / (_B * _C * _H * _W))
    encodings_sum = jnp.zeros((256,), dtype=jnp.float32)
    return (quantized_st, loss, encoding_indices_v, encodings_sum,
            embedding_weight)


# speedup vs baseline: 1.0449x; 1.0449x over previous
import functools

import jax
import jax.numpy as jnp
from jax.experimental import pallas as pl
from jax.experimental.pallas import tpu as pltpu

_B, _C, _H, _W = 16, 64, 32, 32
_P = _H * _W
_K = 1024
_NB = 4
_PT = _NB * _P
_G = _B // _NB


def _vq_body(x_ref, et_ref, e2_ref, q_ref, idx_ref, acc_ref,
             e3_ref, et2_ref):
    g = pl.program_id(0)
    x_cf = x_ref[0]
    e_t = et_ref[...]
    e2 = e2_ref[...]

    @pl.when(g == 0)
    def _build_e3():
        msk = jnp.uint32(0xFFFF0000)
        u = jax.lax.bitcast_convert_type(e_t, jnp.uint32)
        ea_f = jax.lax.bitcast_convert_type(u & msk, jnp.float32)
        r1 = e_t - ea_f
        v = jax.lax.bitcast_convert_type(r1, jnp.uint32)
        eb_f = jax.lax.bitcast_convert_type(v & msk, jnp.float32)
        r2 = r1 - eb_f
        kio = jax.lax.broadcasted_iota(jnp.int32, (1, _K), 1)
        aux = jnp.concatenate(
            [(kio // 256).astype(jnp.float32),
             (kio % 256).astype(jnp.float32),
             jnp.ones((1, _K), jnp.float32)], axis=0)
        e3_ref[...] = jnp.concatenate(
            [ea_f, eb_f, r2, aux], axis=0).astype(jnp.bfloat16)
        et2_ref[...] = e_t + e_t

    e3 = e3_ref[...]

    x_pm = jnp.transpose(x_cf, (0, 2, 1)).reshape(_PT, _C)
    x2 = jnp.sum(x_pm * x_pm, axis=1, keepdims=True)
    xe2 = jax.lax.dot_general(
        x_pm, et2_ref[...], (((1,), (0,)), ((), ())),
        preferred_element_type=jnp.float32)
    distances = (x2 + e2) - xe2

    def _emit(onehot_cp):
        out = jax.lax.dot_general(
            e3, onehot_cp, (((1,), (0,)), ((), ())),
            preferred_element_type=jnp.float32)
        qsum = (out[0:_C] + out[_C:2 * _C]) + out[2 * _C:3 * _C]
        idx_f = out[192:193] * 256.0 + out[193:194]
        idx_i = idx_f[0].astype(jnp.int32)
        for j in range(_NB):
            q_ref[0, j] = qsum[:, j * _P:(j + 1) * _P]
            idx_ref[0, j, :] = idx_i[j * _P:(j + 1) * _P]
        return out[194:195]

    dmin = jnp.min(distances, axis=1, keepdims=True)
    onehot = jnp.where(distances == dmin,
                       1.0, 0.0).astype(jnp.bfloat16)
    nhot = _emit(jnp.transpose(onehot, (1, 0)))

    @pl.when(g == 0)
    def _init():
        acc_ref[...] = jnp.zeros((1, 1), jnp.float32)

    acc_ref[...] += jnp.sum(dmin).reshape(1, 1)


@functools.partial(jax.jit, static_argnames=())
def kernel(inputs, embedding_weight):
    inputs = inputs.astype(jnp.float32)
    x = inputs.reshape(_G, _NB, _C, _P)
    e_t = jnp.transpose(embedding_weight, (1, 0))
    e2 = jnp.sum(embedding_weight.T ** 2, axis=0, keepdims=True)

    q, idx3, acc = pl.pallas_call(
        _vq_body,
        grid=(_G,),
        in_specs=[
            pl.BlockSpec((1, _NB, _C, _P), lambda g: (g, 0, 0, 0)),
            pl.BlockSpec((_C, _K), lambda g: (0, 0)),
            pl.BlockSpec((1, _K), lambda g: (0, 0)),
        ],
        scratch_shapes=[pltpu.VMEM((3 * _C + 3, _K), jnp.bfloat16),
                        pltpu.VMEM((_C, _K), jnp.float32)],
        out_specs=[
            pl.BlockSpec((1, _NB, _C, _P), lambda g: (g, 0, 0, 0)),
            pl.BlockSpec((1, _NB, _P), lambda g: (g, 0, 0)),
            pl.BlockSpec((1, 1), lambda g: (0, 0)),
        ],
        out_shape=[
            jax.ShapeDtypeStruct((_G, _NB, _C, _P), jnp.float32),
            jax.ShapeDtypeStruct((_G, _NB, _P), jnp.int32),
            jax.ShapeDtypeStruct((1, 1), jnp.float32),
        ],
        compiler_params=pltpu.CompilerParams(
            dimension_semantics=("arbitrary",)),
    )(x, e_t, e2)

    quantized_st = q.reshape(_B, _C, _H, _W)
    encoding_indices_v = idx3.reshape(_B, _H, _W)
    loss = acc[0, 0] * (0.25 / (_B * _C * _H * _W))
    encodings_sum = jnp.zeros((256,), dtype=jnp.float32)
    return (quantized_st, loss, encoding_indices_v, encodings_sum,
            embedding_weight)
